# baseline (speedup 1.0000x reference)
import functools

import jax
import jax.numpy as jnp
from jax import lax
from jax.experimental import pallas as pl
from jax.experimental.pallas import tpu as pltpu
from jax.experimental.pallas import tpu_sc as plsc

_LANES = 16
_PW = 128
_NDEEP = 4
_NST = 8
_NSUB = 16


@functools.lru_cache(maxsize=None)
def _make_gather(num_entities, batch, dim, nc, ns):
    nw = nc * ns
    n_panels = (num_entities + _PW - 1) // _PW
    ppw = (n_panels + nw - 1) // nw
    shift = 25
    magic = -(-(1 << shift) // ppw)
    assert all((p * magic) >> shift == p // ppw for p in range(n_panels))
    n_grp = batch // _LANES
    qslots = _NDEEP * ((ppw + 3 + _NDEEP - 1) // _NDEEP)
    mesh = plsc.VectorSubcoreMesh(core_axis_name="c", subcore_axis_name="s")

    @functools.partial(
        pl.kernel,
        out_type=jax.ShapeDtypeStruct((batch * dim + dim,), jnp.float32),
        mesh=mesh,
        scratch_types=(
            [pltpu.VMEM((batch,), jnp.int32)]
            + [pltpu.VMEM((batch + _LANES,), jnp.int32) for _ in range(4)]
            + [pltpu.VMEM((2 * _NSUB,), jnp.int32)]
            + [pltpu.VMEM((dim, _PW), jnp.float32) for _ in range(_NDEEP)]
            + [pltpu.VMEM((dim,), jnp.float32) for _ in range(_NST)]
            + [pltpu.SemaphoreType.DMA for _ in range(_NDEEP)]
            + [pltpu.SemaphoreType.DMA for _ in range(_NST)]
        ),
        compiler_params=pltpu.CompilerParams(
            disable_bounds_checks=True, needs_layout_passes=False
        ),
    )
    def gather_kernel(idx_hbm, tab_hbm, out_hbm, *refs):
        idx_v, hv_v, hb_v, bv_v, bb_v = refs[0:5]
        seg_s = refs[5]
        bufs = refs[6 : 6 + _NDEEP]
        stages = refs[6 + _NDEEP : 6 + _NDEEP + _NST]
        sems = refs[6 + _NDEEP + _NST : 6 + 2 * _NDEEP + _NST]
        osems = refs[6 + 2 * _NDEEP + _NST :]
        wid = lax.axis_index("s") * nc + lax.axis_index("c")
        p0 = wid * ppw
        pltpu.sync_copy(idx_hbm, idx_v)
        iota = lax.iota(jnp.int32, _LANES)
        pad = out_hbm.at[pl.ds(batch * dim, dim)]

        for s in range(_NST):
            pltpu.async_copy(stages[s], pad, osems[s])

        @pl.loop(0, n_grp, init_carry=jnp.int32(0))
        def _scan(g, nh):
            vec = idx_v[pl.ds(g * _LANES, _LANES)]
            own = (vec >> 7) * magic >> shift
            mask = own == wid
            plsc.store_compressed(hv_v.at[pl.ds(nh, _LANES)], vec, mask=mask)
            plsc.store_compressed(
                hb_v.at[pl.ds(nh, _LANES)], g * _LANES + iota, mask=mask
            )
            return nh + plsc.all_reduce_population_count(mask)[0]

        n_hits = _scan
        n_chunks = (n_hits + _LANES - 1) // _LANES

        @pl.loop(0, n_chunks, init_carry=(jnp.int32(0),) * _NSUB)
        def _count(h, cnts):
            sub = ((hv_v[pl.ds(h * _LANES, _LANES)] >> 7) - p0) >> 4
            keep = iota + h * _LANES < n_hits
            return tuple(
                cnts[r] + plsc.all_reduce_population_count((sub == r) & keep)[0]
                for r in range(_NSUB)
            )

        counts = _count
        base = jnp.int32(0)
        bases = []
        for r in range(_NSUB):
            bases.append(base)
            base = base + counts[r]
        bvec = jnp.zeros((_LANES,), jnp.int32)
        lvec = jnp.zeros((_LANES,), jnp.int32)
        for r in range(_NSUB):
            bvec = jnp.where(iota == r, jnp.full((_LANES,), bases[r]), bvec)
            lvec = jnp.where(iota == r, jnp.full((_LANES,), counts[r]), lvec)
        seg_s[pl.ds(0, _LANES)] = bvec
        seg_s[pl.ds(_LANES, _LANES)] = lvec

        for r in range(_NSUB):

            @pl.loop(0, n_chunks, init_carry=bases[r])
            def _fill(h, off, r=r):
                hv = hv_v[pl.ds(h * _LANES, _LANES)]
                hb = hb_v[pl.ds(h * _LANES, _LANES)]
                sub = ((hv >> 7) - p0) >> 4
                pm = (sub == r) & (iota + h * _LANES < n_hits)
                plsc.store_compressed(bv_v.at[pl.ds(off, _LANES)], hv, mask=pm)
                plsc.store_compressed(bb_v.at[pl.ds(off, _LANES)], hb, mask=pm)
                return off + plsc.all_reduce_population_count(pm)[0]

        def fetchable(q):
            return (p0 + q < n_panels) & (q < ppw + 3)

        def start(q, slot):
            @pl.when(fetchable(q))
            def _():
                off = pl.multiple_of((p0 + q) * _PW, _PW)
                pltpu.async_copy(
                    tab_hbm.at[:, pl.ds(off, _PW)], bufs[slot], sems[slot]
                )

        for s in range(_NDEEP):
            start(jnp.int32(s), s)

        @pl.loop(0, qslots // _NDEEP)
        def _panels(t):
            for s in range(_NDEEP):
                q = t * _NDEEP + s
                gp = p0 + q
                sub_ix = jnp.full((_LANES,), q >> 4, jnp.int32)
                sb = plsc.load_gather(seg_s, [sub_ix])[0]
                sl = plsc.load_gather(seg_s, [sub_ix + _NSUB])[0]

                @pl.when(fetchable(q))
                def _():
                    pltpu.make_async_copy(
                        tab_hbm.at[:, pl.ds(0, _PW)], bufs[s], sems[s]
                    ).wait()

                @pl.loop(0, (sl + _LANES - 1) // _LANES, init_carry=jnp.int32(0))
                def _pscan(h, c2, sb=sb, sl=sl, gp=gp):
                    hv = bv_v[pl.ds(sb + h * _LANES, _LANES)]
                    hb = bb_v[pl.ds(sb + h * _LANES, _LANES)]
                    pm = ((hv >> 7) == gp) & (iota + h * _LANES < sl)
                    plsc.store_compressed(hv_v.at[pl.ds(c2, _LANES)], hv, mask=pm)
                    plsc.store_compressed(hb_v.at[pl.ds(c2, _LANES)], hb, mask=pm)
                    return c2 + plsc.all_reduce_population_count(pm)[0]

                c2 = _pscan

                @pl.loop(0, (c2 + _LANES - 1) // _LANES)
                def _extract(e, c2=c2, s=s):
                    ev = hv_v[pl.ds(e * _LANES, _LANES)]
                    eb = hb_v[pl.ds(e * _LANES, _LANES)]
                    for l in range(_LANES):

                        @pl.when(e * _LANES + l < c2)
                        def _():
                            st = stages[l % _NST]
                            pltpu.make_async_copy(st, pad, osems[l % _NST]).wait()
                            lane = jnp.full((_LANES,), ev[l] & (_PW - 1), jnp.int32)
                            for k in range(dim // _LANES):
                                st[pl.ds(k * _LANES, _LANES)] = plsc.load_gather(
                                    bufs[s], [iota + (k * _LANES), lane]
                                )
                            pltpu.async_copy(
                                st,
                                out_hbm.at[pl.ds(eb[l] * dim, dim)],
                                osems[l % _NST],
                            )

                start(q + _NDEEP, s)

        for s in range(_NST):
            pltpu.make_async_copy(stages[s], pad, osems[s]).wait()

    return gather_kernel


def kernel(elements, entity_embeddings):
    (batch,) = elements.shape
    num_entities, dim = entity_embeddings.shape
    info = plsc.get_sparse_core_info()
    fn = _make_gather(num_entities, batch, dim, info.num_cores, info.num_subcores)
    flat = fn(elements, entity_embeddings.T)
    return flat[: batch * dim].reshape(batch, dim)


# speedup vs baseline: 1.4381x; 1.4381x over previous
import functools

import jax
import jax.numpy as jnp
from jax import lax
from jax.experimental import pallas as pl
from jax.experimental.pallas import tpu as pltpu
from jax.experimental.pallas import tpu_sc as plsc

_LANES = 16
_NDEEP = 8
_NQ = 4


@functools.lru_cache(maxsize=None)
def _make_gather(num_entities, batch, dim, nc, ns):
    nw = nc * ns
    b_per_w = batch // nw
    n_grp = b_per_w // _LANES
    grp_per_q = n_grp // _NQ
    mesh = plsc.VectorSubcoreMesh(core_axis_name="c", subcore_axis_name="s")

    @functools.partial(
        pl.kernel,
        out_type=jax.ShapeDtypeStruct((dim, batch), jnp.float32),
        mesh=mesh,
        scratch_types=(
            [pltpu.VMEM((b_per_w,), jnp.int32)]
            + [pltpu.VMEM((dim, 128), jnp.float32) for _ in range(_NDEEP)]
            + [pltpu.VMEM((dim, 128), jnp.float32) for _ in range(_NQ)]
            + [pltpu.SemaphoreType.DMA for _ in range(_NDEEP)]
            + [pltpu.SemaphoreType.DMA]
        ),
        compiler_params=pltpu.CompilerParams(
            disable_bounds_checks=True, needs_layout_passes=False
        ),
    )
    def gather_kernel(idx_hbm, tab_hbm, out_hbm, *refs):
        idx_v = refs[0]
        bufs = refs[1 : 1 + _NDEEP]
        qbufs = refs[1 + _NDEEP : 1 + _NDEEP + _NQ]
        sems = refs[1 + _NDEEP + _NQ : 1 + 2 * _NDEEP + _NQ]
        osem = refs[1 + 2 * _NDEEP + _NQ]
        wid = lax.axis_index("s") * nc + lax.axis_index("c")
        base = wid * b_per_w
        pltpu.sync_copy(idx_hbm.at[pl.ds(base, b_per_w)], idx_v)
        iota = lax.iota(jnp.int32, _LANES)

        def start(v, par):
            off = pl.multiple_of((v >> 7) * 128, 128)
            pltpu.async_copy(tab_hbm.at[:, pl.ds(off, 128)], bufs[par], sems[par])

        def finish(gl, l, v, par, qbuf):
            pltpu.make_async_copy(
                tab_hbm.at[:, pl.ds(0, 128)], bufs[par], sems[par]
            ).wait()
            lane = jnp.full((_LANES,), v & 127, jnp.int32)
            col = jnp.full((_LANES,), gl * _LANES + l, jnp.int32)
            for k in range(dim // _LANES):
                vals = plsc.load_gather(bufs[par], [iota + (k * _LANES), lane])
                plsc.store_scatter(qbuf, [iota + (k * _LANES), col], vals)

        vec0 = idx_v[pl.ds(0, _LANES)]
        for l in range(_NDEEP):
            start(vec0[l], l)

        vec = vec0
        for q in range(_NQ):

            @pl.loop(0, grp_per_q, init_carry=vec)
            def _grp(g, vec, q=q):
                gg = q * grp_per_q + g
                nxt_off = jnp.minimum((gg + 1) * _LANES, b_per_w - _LANES)
                vec_n = idx_v[pl.ds(nxt_off, _LANES)]
                for l in range(_LANES):
                    finish(g, l, vec[l], l % _NDEEP, qbufs[q])
                    if l < _LANES - _NDEEP:
                        start(vec[l + _NDEEP], (l + _NDEEP) % _NDEEP)
                    else:

                        @pl.when(gg < n_grp - 1)
                        def _():
                            start(vec_n[l + _NDEEP - _LANES], (l + _NDEEP) % _NDEEP)

                return vec_n

            vec = _grp
            pltpu.async_copy(
                qbufs[q],
                out_hbm.at[:, pl.ds(base + q * 128, 128)],
                osem,
            )

        for q in range(_NQ):
            pltpu.make_async_copy(
                qbufs[q], out_hbm.at[:, pl.ds(base + q * 128, 128)], osem
            ).wait()

    return gather_kernel


def kernel(elements, entity_embeddings):
    (batch,) = elements.shape
    num_entities, dim = entity_embeddings.shape
    info = plsc.get_sparse_core_info()
    fn = _make_gather(num_entities, batch, dim, info.num_cores, info.num_subcores)
    out_t = fn(elements, entity_embeddings.T)
    return out_t.T
